# Initial kernel scaffold; baseline (speedup 1.0000x reference)
import functools

import jax
import jax.numpy as jnp
from jax import lax
from jax.experimental import pallas as pl
from jax.experimental.pallas import tpu as pltpu
from jax.experimental.pallas import tpu_sc as plsc

NC = 2
NS = 16
NW = NC * NS
L = 16


def _make_spmm(N, D, E):
    EW = E // NW
    C = 125
    NCHUNK = EW // C
    RPS = N // NS
    ZB = RPS // 5
    mesh = plsc.VectorSubcoreMesh(
        core_axis_name="c", subcore_axis_name="s", num_cores=NC, num_subcores=NS
    )

    @functools.partial(
        pl.kernel,
        out_type=jax.ShapeDtypeStruct((NC, N, D), jnp.float32),
        mesh=mesh,
        scratch_types=[
            pltpu.VMEM((NCHUNK, C), jnp.int32),
            pltpu.VMEM((NCHUNK, C), jnp.int32),
            pltpu.VMEM((NCHUNK, C), jnp.float32),
            pltpu.VMEM((C, D), jnp.float32),
            pltpu.VMEM((C, D), jnp.float32),
            pltpu.VMEM_SHARED((N, D), jnp.float32),
            pltpu.SemaphoreType.DMA,
            pltpu.SemaphoreType.DMA,
        ],
    )
    def spmm(ego_hbm, src_hbm, dst_hbm, val_hbm, out_hbm,
             srcv, dstv, valv, rows0, rows1, acc, sem0, sem1):
        cid = lax.axis_index("c")
        sid = lax.axis_index("s")
        w = cid * NS + sid
        rows = (rows0, rows1)
        sems = (sem0, sem1)

        def zero_body(r, carry):
            for j in range(D // L):
                rows0[r, pl.ds(j * L, L)] = jnp.zeros((L,), jnp.float32)
            return carry
        lax.fori_loop(0, C, zero_body, 0)
        for z in range(RPS // ZB):
            pltpu.sync_copy(rows0.at[pl.ds(0, ZB)],
                            acc.at[pl.ds(sid * RPS + z * ZB, ZB)])
        plsc.subcore_barrier()

        pltpu.sync_copy(src_hbm.at[w], srcv)
        pltpu.sync_copy(dst_hbm.at[w], dstv)
        pltpu.sync_copy(val_hbm.at[w], valv)

        def gather_start(ci, b):
            pltpu.make_async_copy(
                ego_hbm.at[srcv.at[ci]], rows[b], sems[b]).start()

        def gather_wait(ci, b):
            pltpu.make_async_copy(
                ego_hbm.at[srcv.at[ci]], rows[b], sems[b]).wait()

        gather_start(0, 0)

        UNROLL = 5

        def do_chunk(ci, b):
            gather_wait(ci, b)
            rb = rows[b]

            def scale_body(g, carry):
                for u in range(UNROLL):
                    e = g * UNROLL + u
                    vbc = plsc.load_gather(
                        valv,
                        [jnp.full((L,), ci, jnp.int32),
                         jnp.full((L,), e, jnp.int32)],
                    )
                    for j in range(D // L):
                        rb[e, pl.ds(j * L, L)] = rb[e, pl.ds(j * L, L)] * vbc
                return carry
            lax.fori_loop(0, C // UNROLL, scale_body, 0)
            pltpu.sync_copy(rb, acc.at[dstv.at[ci]], add=True)

        def outer(c2, carry):
            for b in range(2):
                ci = c2 * 2 + b

                @pl.when(ci + 1 < NCHUNK)
                def _():
                    gather_start(ci + 1, (b + 1) % 2)

                do_chunk(ci, b)
            return carry
        lax.fori_loop(0, NCHUNK // 2, outer, 0)

        plsc.subcore_barrier()
        pltpu.sync_copy(acc.at[pl.ds(sid * RPS, RPS)],
                        out_hbm.at[cid, pl.ds(sid * RPS, RPS)])

    return spmm


def _layer_tc_body(a0, a1, eg, wg, bg, wb, bb, eo, no):
    side = a0[...] + a1[...]
    e = eg[...]
    s = jnp.dot(side, wg[...], preferred_element_type=jnp.float32) + bg[...]
    bi = jnp.dot(e * side, wb[...], preferred_element_type=jnp.float32) + bb[...]
    act = s + bi
    act = jnp.where(act >= 0, act, 0.2 * act)
    eo[...] = act
    nn = jnp.sqrt(jnp.sum(act * act, axis=1, keepdims=True))
    no[...] = act / jnp.maximum(nn, 1e-12)


def _make_layer_tc(N, D, R=1000):
    grid = N // R
    row_spec = pl.BlockSpec((R, D), lambda i: (i, 0))
    full = pl.BlockSpec((D, D), lambda i: (0, 0))
    bias = pl.BlockSpec((1, D), lambda i: (0, 0))
    return pl.pallas_call(
        _layer_tc_body,
        grid=(grid,),
        in_specs=[row_spec, row_spec, row_spec, full, bias, full, bias],
        out_specs=[row_spec, row_spec],
        out_shape=[
            jax.ShapeDtypeStruct((N, D), jnp.float32),
            jax.ShapeDtypeStruct((N, D), jnp.float32),
        ],
    )


def _proj_tc_body(e0, n1, n2, n3, wp, bp, out):
    acc = jnp.dot(e0[...], wp[0], preferred_element_type=jnp.float32)
    acc += jnp.dot(n1[...], wp[1], preferred_element_type=jnp.float32)
    acc += jnp.dot(n2[...], wp[2], preferred_element_type=jnp.float32)
    acc += jnp.dot(n3[...], wp[3], preferred_element_type=jnp.float32)
    out[...] = acc + bp[...]


def _make_proj_tc(N, D, OUT, R=1000):
    grid = N // R
    row_spec = pl.BlockSpec((R, D), lambda i: (i, 0))
    wspec = pl.BlockSpec((4, D, OUT), lambda i: (0, 0, 0))
    bspec = pl.BlockSpec((1, OUT), lambda i: (0, 0))
    out_spec = pl.BlockSpec((R, OUT), lambda i: (i, 0))
    return pl.pallas_call(
        _proj_tc_body,
        grid=(grid,),
        in_specs=[row_spec, row_spec, row_spec, row_spec, wspec, bspec],
        out_specs=out_spec,
        out_shape=jax.ShapeDtypeStruct((N, OUT), jnp.float32),
    )


def kernel(ego_embeddings, adj_vals, dst, src, W_gc, b_gc, W_bi, b_bi,
           W_proj, b_proj):
    N, D = ego_embeddings.shape
    E = src.shape[0]
    NLAYERS = W_gc.shape[0]
    OUT = W_proj.shape[1]
    EW = E // NW
    C = 125
    NCHUNK = EW // C

    src_r = src.astype(jnp.int32).reshape(NW, NCHUNK, C)
    dst_r = dst.astype(jnp.int32).reshape(NW, NCHUNK, C)
    val_r = adj_vals.reshape(NW, NCHUNK, C)

    spmm = _make_spmm(N, D, E)
    layer_tc = _make_layer_tc(N, D)
    proj_tc = _make_proj_tc(N, D, OUT)

    ego = ego_embeddings
    parts = [ego]
    for k in range(NLAYERS):
        acc = spmm(ego, src_r, dst_r, val_r)
        ego, nrm = layer_tc(acc[0], acc[1], ego, W_gc[k], b_gc[k],
                            W_bi[k], b_bi[k])
        parts.append(nrm)

    return proj_tc(parts[0], parts[1], parts[2], parts[3],
                   W_proj.reshape(NLAYERS + 1, D, OUT),
                   b_proj.reshape(1, OUT))


# speedup vs baseline: 5.2868x; 5.2868x over previous
import functools

import jax
import jax.numpy as jnp
from jax import lax
from jax.experimental import pallas as pl
from jax.experimental.pallas import tpu as pltpu
from jax.experimental.pallas import tpu_sc as plsc

NC = 2
NS = 16
NW = NC * NS
L = 16
C = 80


def _make_spmm(N, D, NCT):
    RPS = (N // NS) // 8 * 8
    LAST = N - (NS - 1) * RPS
    TAIL = RPS - (RPS // C) * C
    mesh = plsc.VectorSubcoreMesh(
        core_axis_name="c", subcore_axis_name="s", num_cores=NC, num_subcores=NS
    )

    @functools.partial(
        pl.kernel,
        out_type=[
            jax.ShapeDtypeStruct((N, D), jnp.float32),
            jax.ShapeDtypeStruct((N, D), jnp.float32),
        ],
        mesh=mesh,
        compiler_params=pltpu.CompilerParams(needs_layout_passes=False),
        scratch_types=[
            pltpu.VMEM((3, C), jnp.int32),
            pltpu.VMEM((3, C), jnp.int32),
            pltpu.VMEM((C, D), jnp.float32),
            pltpu.VMEM((C, D), jnp.float32),
            pltpu.VMEM_SHARED((N, D), jnp.float32),
            pltpu.SemaphoreType.DMA,
            pltpu.SemaphoreType.DMA,
            pltpu.SemaphoreType.DMA,
            pltpu.SemaphoreType.DMA,
        ],
    )
    def spmm(ego, edata, out_l, out_r,
             eb0, eb1, rows0, rows1, acc,
             semi0, semi1, semr0, semr1):
        cid = lax.axis_index("c")
        sid = lax.axis_index("s")
        w = cid * NS + sid
        eb = (eb0, eb1)
        rows = (rows0, rows1)
        semi = (semi0, semi1)
        semr = (semr0, semr1)

        def zero_body(r, carry):
            for j in range(D // L):
                rows0[r, pl.ds(j * L, L)] = jnp.zeros((L,), jnp.float32)
            return carry
        lax.fori_loop(0, C, zero_body, 0)
        for z in range(RPS // C):
            pltpu.sync_copy(rows0, acc.at[pl.ds(sid * RPS + z * C, C)])
        if TAIL:
            pltpu.sync_copy(rows0.at[pl.ds(0, TAIL)],
                            acc.at[pl.ds(sid * RPS + (RPS // C) * C, TAIL)])

        @pl.when(sid == NS - 1)
        def _():
            pltpu.sync_copy(rows0.at[pl.ds(0, LAST - RPS)],
                            acc.at[pl.ds(NS * RPS, LAST - RPS)])
        plsc.subcore_barrier()

        def idx_start(ci, b):
            pltpu.make_async_copy(edata.at[w, ci], eb[b], semi[b]).start()

        def idx_wait(ci, b):
            pltpu.make_async_copy(edata.at[w, ci], eb[b], semi[b]).wait()

        def gather_start(b):
            pltpu.make_async_copy(
                ego.at[eb[b].at[0]], rows[b], semr[b]).start()

        def gather_wait(b):
            pltpu.make_async_copy(
                ego.at[eb[b].at[0]], rows[b], semr[b]).wait()

        def scale(b):
            rb = rows[b]
            ebb = eb[b]

            def scale_body(g, carry):
                vals16 = plsc.bitcast(ebb[2, pl.ds(g * L, L)], jnp.float32)
                for u in range(L):
                    vbc = lax.gather(
                        vals16,
                        jnp.full((L, 1), u, jnp.int32),
                        lax.GatherDimensionNumbers(
                            offset_dims=(), collapsed_slice_dims=(0,),
                            start_index_map=(0,)),
                        (1,),
                        mode=lax.GatherScatterMode.PROMISE_IN_BOUNDS,
                        indices_are_sorted=True)
                    e = g * L + u
                    for j in range(D // L):
                        rb[e, pl.ds(j * L, L)] = rb[e, pl.ds(j * L, L)] * vbc
                return carry
            lax.fori_loop(0, C // L, scale_body, 0)

        idx_start(0, 0)
        idx_wait(0, 0)
        gather_start(0)
        idx_start(1, 1)

        def do_iter(ci, b):
            nb = (b + 1) % 2

            @pl.when(ci + 1 < NCT)
            def _():
                idx_wait(ci + 1, nb)
                gather_start(nb)

            gather_wait(b)
            scale(b)
            pltpu.sync_copy(rows[b], acc.at[eb[b].at[1]], add=True)

            @pl.when(ci + 2 < NCT)
            def _():
                idx_start(ci + 2, b)

        def outer(c2, carry):
            for bb in range(2):
                do_iter(c2 * 2 + bb, bb)
            return carry
        lax.fori_loop(0, NCT // 2, outer, 0)

        plsc.subcore_barrier()

        def write_out(dst_hbm):
            @pl.when(sid < NS - 1)
            def _():
                pltpu.sync_copy(acc.at[pl.ds(sid * RPS, RPS)],
                                dst_hbm.at[pl.ds(sid * RPS, RPS)])

            @pl.when(sid == NS - 1)
            def _():
                pltpu.sync_copy(acc.at[pl.ds((NS - 1) * RPS, LAST)],
                                dst_hbm.at[pl.ds((NS - 1) * RPS, LAST)])

        @pl.when(cid == 0)
        def _():
            write_out(out_l)

        @pl.when(cid == 1)
        def _():
            write_out(out_r)

    return spmm


def _layer_tc_body(a0, a1, eg, wg, bg, wb, bb, eo, no):
    side = a0[...] + a1[...]
    e = eg[...]
    s = jnp.dot(side, wg[...], preferred_element_type=jnp.float32) + bg[...]
    bi = jnp.dot(e * side, wb[...], preferred_element_type=jnp.float32) + bb[...]
    act = s + bi
    act = jnp.where(act >= 0, act, 0.2 * act)
    eo[...] = act
    nn = jnp.sqrt(jnp.sum(act * act, axis=1, keepdims=True))
    no[...] = act / jnp.maximum(nn, 1e-12)


def _make_layer_tc(N, D, R=1000):
    grid = N // R
    row_spec = pl.BlockSpec((R, D), lambda i: (i, 0))
    wspec = pl.BlockSpec((D, D), lambda i: (0, 0))
    bias = pl.BlockSpec((1, D), lambda i: (0, 0))
    return pl.pallas_call(
        _layer_tc_body,
        grid=(grid,),
        in_specs=[row_spec, row_spec, row_spec, wspec, bias, wspec, bias],
        out_specs=[row_spec, row_spec],
        out_shape=[
            jax.ShapeDtypeStruct((N, D), jnp.float32),
            jax.ShapeDtypeStruct((N, D), jnp.float32),
        ],
    )


def _proj_tc_body(e0, n1, n2, n3, wp, bp, out):
    acc = jnp.dot(e0[...], wp[0], preferred_element_type=jnp.float32)
    acc += jnp.dot(n1[...], wp[1], preferred_element_type=jnp.float32)
    acc += jnp.dot(n2[...], wp[2], preferred_element_type=jnp.float32)
    acc += jnp.dot(n3[...], wp[3], preferred_element_type=jnp.float32)
    out[...] = acc + bp[...]


def _make_proj_tc(N, D, OUT, R=1000):
    grid = N // R
    row_spec = pl.BlockSpec((R, D), lambda i: (i, 0))
    wspec = pl.BlockSpec((4, D, OUT), lambda i: (0, 0, 0))
    bspec = pl.BlockSpec((1, OUT), lambda i: (0, 0))
    out_spec = pl.BlockSpec((R, OUT), lambda i: (i, 0))
    return pl.pallas_call(
        _proj_tc_body,
        grid=(grid,),
        in_specs=[row_spec, row_spec, row_spec, row_spec, wspec, bspec],
        out_specs=out_spec,
        out_shape=jax.ShapeDtypeStruct((N, OUT), jnp.float32),
    )


def kernel(ego_embeddings, adj_vals, dst, src, W_gc, b_gc, W_bi, b_bi,
           W_proj, b_proj):
    N, D = ego_embeddings.shape
    E = src.shape[0]
    NLAYERS = W_gc.shape[0]
    OUT = W_proj.shape[1]
    EWT = E // NW
    NCT = -(-EWT // C)
    NCT = -(-NCT // 2) * 2
    PAD = NCT * C - EWT

    def prep(x):
        x = x.reshape(NW, EWT)
        if PAD:
            x = jnp.concatenate(
                [x, jnp.zeros((NW, PAD), x.dtype)], axis=1)
        return x.reshape(NW, NCT, C)

    src_p = prep(src.astype(jnp.int32))
    dst_p = prep(dst.astype(jnp.int32))
    val_p = prep(lax.bitcast_convert_type(adj_vals, jnp.int32))
    edata = jnp.stack([src_p, dst_p, val_p], axis=2)

    spmm = _make_spmm(N, D, NCT)
    layer_tc = _make_layer_tc(N, D)
    proj_tc = _make_proj_tc(N, D, OUT)

    ego = ego_embeddings
    parts = [ego_embeddings]
    for k in range(NLAYERS):
        al, ar = spmm(ego, edata)
        ego, nrm = layer_tc(al, ar, ego, W_gc[k], b_gc[k], W_bi[k], b_bi[k])
        parts.append(nrm)

    return proj_tc(parts[0], parts[1], parts[2], parts[3],
                   W_proj.reshape(NLAYERS + 1, D, OUT),
                   b_proj.reshape(1, OUT))
